# Initial kernel scaffold; baseline (speedup 1.0000x reference)
import jax
import jax.numpy as jnp
from jax.experimental import pallas as pl


def kernel(edges, features, emb_users, emb_known, W0, b0, W1, b1, W2, b2):
    raise NotImplementedError("write your pallas kernel here")


# speedup vs baseline: 21.2429x; 21.2429x over previous
import functools

import jax
import jax.numpy as jnp
from jax import lax
from jax.experimental import pallas as pl
from jax.experimental.pallas import tpu as pltpu
from jax.experimental.pallas import tpu_sc as plsc

_N = 10000
_NP = 10240
_E = 320000
_EP = 327680
_DIN = 128
_H = 32
_W8 = 8
_CH = 128
_NCHUNK = _EP // _CH
_CPT = _NCHUNK // 32
_GCH = 64
_NGCHUNK = _NP // _GCH
_GPT = _NGCHUNK // 32
_RPT = _NP // 16

_mesh = plsc.VectorSubcoreMesh(core_axis_name="c", subcore_axis_name="s",
                               num_cores=2, num_subcores=16)


def _mm_body(x_ref, w_ref, o_ref):
    o_ref[...] = jnp.dot(x_ref[...], w_ref[...],
                         preferred_element_type=jnp.float32)


_mm0 = pl.pallas_call(
    _mm_body,
    grid=(5,),
    in_specs=[
        pl.BlockSpec((_NP // 5, _DIN), lambda i: (i, 0)),
        pl.BlockSpec((_DIN, _H), lambda i: (0, 0)),
    ],
    out_specs=pl.BlockSpec((_NP // 5, _H), lambda i: (i, 0)),
    out_shape=jax.ShapeDtypeStruct((_NP, _H), jnp.float32),
)


def _k2_body(deg_ref, xwg_ref, tw_ref, f1_ref, dinv_ref, y_ref):
    deg = deg_ref[0] + deg_ref[1] + 1.0
    dinv = lax.rsqrt(deg)
    dinv_ref[...] = dinv
    ew0 = tw_ref[_N, :]
    ew1 = tw_ref[_N + 1, :]
    sel = jnp.where(f1_ref[...] == 1, ew1[None, :], ew0[None, :])
    y_ref[...] = (xwg_ref[...] + sel) * dinv


_k2 = pl.pallas_call(
    _k2_body,
    out_shape=(
        jax.ShapeDtypeStruct((_NP, 1), jnp.float32),
        jax.ShapeDtypeStruct((_NP, _H), jnp.float32),
    ),
)


def _k3_body(p_ref, y_ref, dinv_ref, b0_ref, w2_ref, z8_ref):
    dinv = dinv_ref[...]
    x1 = jnp.maximum(dinv * (p_ref[0] + p_ref[1] + y_ref[...]) + b0_ref[...],
                     0.0)
    z = jnp.dot(x1, w2_ref[...], preferred_element_type=jnp.float32)
    z8_ref[...] = z * dinv


_k3 = pl.pallas_call(
    _k3_body,
    out_shape=jax.ShapeDtypeStruct((_NP, _W8), jnp.float32),
)


def _k4_body(q_ref, z8_ref, dinv_ref, b2_ref, o_ref):
    o_ref[...] = dinv_ref[...] * (q_ref[0] + q_ref[1] + z8_ref[...]) \
        + b2_ref[...]


_k4 = pl.pallas_call(
    _k4_body,
    out_shape=jax.ShapeDtypeStruct((_NP, _W8), jnp.float32),
)


def _s1_body(dst_hbm, f0_hbm, tw_hbm, zdeg_hbm,
             deg_out, xwg_out,
             deg_sp, tw_sp, idxbuf, onesbuf, nidx, gbuf, sem):
    c = lax.axis_index("c")
    s = lax.axis_index("s")
    w = c * 16 + s
    pltpu.sync_copy(zdeg_hbm.at[pl.ds(s * _RPT, _RPT)],
                    deg_sp.at[pl.ds(s * _RPT, _RPT)])
    pltpu.sync_copy(tw_hbm.at[pl.ds(s * _RPT, _RPT)],
                    tw_sp.at[pl.ds(s * _RPT, _RPT)])
    for i in range(_CH // 16):
        onesbuf[pl.ds(16 * i, 16)] = jnp.ones((16,), jnp.float32)
    plsc.subcore_barrier()

    base = c * (_NCHUNK // 2) + s * _CPT

    def deg_step(j, carry):
        pltpu.sync_copy(dst_hbm.at[base + j], idxbuf)
        pltpu.sync_copy(onesbuf, deg_sp.at[idxbuf], add=True)
        return carry

    lax.fori_loop(0, _CPT, deg_step, 0)

    def gather_step(j, carry):
        r = w * _GPT + j
        pltpu.sync_copy(f0_hbm.at[r], nidx)
        pltpu.async_copy(tw_sp.at[nidx], gbuf, sem).wait()
        pltpu.sync_copy(gbuf, xwg_out.at[pl.ds(r * _GCH, _GCH)])
        return carry

    lax.fori_loop(0, _GPT, gather_step, 0)
    plsc.subcore_barrier()
    pltpu.sync_copy(deg_sp.at[pl.ds(s * _RPT, _RPT)],
                    deg_out.at[c, pl.ds(s * _RPT, _RPT)])


_s1 = pl.kernel(
    _s1_body,
    out_type=(
        jax.ShapeDtypeStruct((2, _NP), jnp.float32),
        jax.ShapeDtypeStruct((_NP, _H), jnp.float32),
    ),
    mesh=_mesh,
    scratch_types=[
        pltpu.VMEM_SHARED((_NP,), jnp.float32),
        pltpu.VMEM_SHARED((_NP, _H), jnp.float32),
        pltpu.VMEM((_CH,), jnp.int32),
        pltpu.VMEM((_CH,), jnp.float32),
        pltpu.VMEM((_GCH,), jnp.int32),
        pltpu.VMEM((_GCH, _H), jnp.float32),
        pltpu.SemaphoreType.DMA,
    ],
)


def _agg_body(tab_hbm, src_hbm, dst_hbm, zeros_hbm,
              part_out,
              tab_sp, acc_sp, sidx, didx, buf, sem):
    c = lax.axis_index("c")
    s = lax.axis_index("s")
    pltpu.sync_copy(tab_hbm.at[pl.ds(s * _RPT, _RPT)],
                    tab_sp.at[pl.ds(s * _RPT, _RPT)])
    pltpu.sync_copy(zeros_hbm.at[pl.ds(s * _RPT, _RPT)],
                    acc_sp.at[pl.ds(s * _RPT, _RPT)])
    plsc.subcore_barrier()

    base = c * (_NCHUNK // 2) + s * _CPT

    def step(j, carry):
        pltpu.sync_copy(src_hbm.at[base + j], sidx)
        pltpu.sync_copy(dst_hbm.at[base + j], didx)
        pltpu.async_copy(tab_sp.at[sidx], buf, sem).wait()
        pltpu.sync_copy(buf, acc_sp.at[didx], add=True)
        return carry

    lax.fori_loop(0, _CPT, step, 0)
    plsc.subcore_barrier()
    pltpu.sync_copy(acc_sp.at[pl.ds(s * _RPT, _RPT)],
                    part_out.at[c, pl.ds(s * _RPT, _RPT)])


def _make_agg(width):
    return pl.kernel(
        _agg_body,
        out_type=jax.ShapeDtypeStruct((2, _NP, width), jnp.float32),
        mesh=_mesh,
        scratch_types=[
            pltpu.VMEM_SHARED((_NP, width), jnp.float32),
            pltpu.VMEM_SHARED((_NP, width), jnp.float32),
            pltpu.VMEM((_CH,), jnp.int32),
            pltpu.VMEM((_CH,), jnp.int32),
            pltpu.VMEM((_CH, width), jnp.float32),
            pltpu.SemaphoreType.DMA,
        ],
    )


_agg32 = _make_agg(_H)
_agg8 = _make_agg(_W8)


@jax.jit
def kernel(edges, features, emb_users, emb_known, W0, b0, W1, b1, W2, b2):
    f32 = jnp.float32
    npad = _NP - _N
    t_all = jnp.concatenate(
        [emb_users, emb_known, jnp.zeros((npad - 2, _DIN), f32)], axis=0)

    f0 = features[:, 0].astype(jnp.int32)
    f1 = features[:, 1].astype(jnp.int32)
    f0p = jnp.concatenate(
        [f0, jnp.arange(npad, dtype=jnp.int32) % _N]).reshape(_NGCHUNK, _GCH)
    f1p = jnp.concatenate(
        [f1, jnp.zeros((npad,), jnp.int32)]).reshape(_NP, 1)

    epad = _EP - _E
    dummy = _N + (jnp.arange(epad, dtype=jnp.int32) % npad)
    srcp = jnp.concatenate(
        [edges[0].astype(jnp.int32), dummy]).reshape(_NCHUNK, _CH)
    dstp = jnp.concatenate(
        [edges[1].astype(jnp.int32), dummy]).reshape(_NCHUNK, _CH)

    zdeg = jnp.zeros((_NP,), f32)
    z32 = jnp.zeros((_NP, _H), f32)
    z8 = jnp.zeros((_NP, _W8), f32)
    w2p = jnp.concatenate([W2, jnp.zeros((_H, _W8 - 2), f32)], axis=1)
    b0r = b0.reshape(1, _H)
    b2p = jnp.concatenate([b2, jnp.zeros((_W8 - 2,), f32)]).reshape(1, _W8)

    tw = _mm0(t_all, W0)
    degp, xwg = _s1(dstp, f0p, tw, zdeg)
    dinv, y = _k2(degp.reshape(2, _NP, 1), xwg, tw, f1p)
    p = _agg32(y, srcp, dstp, z32)
    z8a = _k3(p, y, dinv, b0r, w2p)
    q = _agg8(z8a, srcp, dstp, z8)
    out8 = _k4(q, z8a, dinv, b2p)
    return out8[:_N, :2]
